# Initial kernel scaffold; baseline (speedup 1.0000x reference)
import jax
import jax.numpy as jnp
from jax.experimental import pallas as pl


def kernel(edge_index, graph_ids, W1, b1, W2, b2, Wc, bc):
    raise NotImplementedError("write your pallas kernel here")


# speedup vs baseline: 5.2860x; 5.2860x over previous
import functools
import jax
import jax.numpy as jnp
from jax import lax
from jax.experimental import pallas as pl
from jax.experimental.pallas import tpu as pltpu
from jax.experimental.pallas import tpu_sc as plsc

N_NODES = 10000
N_EDGES = 320000
HIDDEN = 128
N_CLASSES = 10
N_GRAPHS = 128

NP = 10240
NC = 2
NS = 16
NPT = NP // NS
CHUNK = 128
CH_ALL = 158
EP = NC * NS * 79 * CHUNK
CH_HALF = 79
EPH = EP // 2


def _rsqrt16(x):
    xs = jnp.maximum(x, 1.0)
    sc = jnp.where(xs >= 65536.0, 256.0, jnp.where(xs >= 256.0, 16.0, 1.0))
    xp = xs / (sc * sc)
    s = 0.09 * xp + 1.6
    for _ in range(5):
        s = 0.5 * (s + xp / s)
    return jnp.where(x > 0.0, 1.0 / (sc * s), 0.0)


def _sc_pre_body(src_h, dst_h, z1_h, araw_h, nin_h, nout_h,
                 din_s, dout_s, g_s, a_s,
                 src_v, dst_v, ones_v, vals_v,
                 b0_v, b1_v, b2_v, b3_v):
    c = lax.axis_index("c")
    t = lax.axis_index("s")
    nb = t * NPT

    pltpu.sync_copy(z1_h.at[pl.ds(nb, NPT)], din_s.at[pl.ds(nb, NPT)])
    pltpu.sync_copy(z1_h.at[pl.ds(nb, NPT)], dout_s.at[pl.ds(nb, NPT)])
    pltpu.sync_copy(z1_h.at[pl.ds(nb, NPT)], a_s.at[pl.ds(nb, NPT)])
    for i in range(CHUNK // 16):
        ones_v[pl.ds(i * 16, 16)] = jnp.ones((16,), jnp.float32)
    plsc.subcore_barrier()

    def p0(j, _):
        e0 = t * (CH_ALL * CHUNK) + j * CHUNK
        pltpu.sync_copy(src_h.at[pl.ds(e0, CHUNK)], src_v)
        pltpu.sync_copy(dst_h.at[pl.ds(e0, CHUNK)], dst_v)
        pltpu.sync_copy(ones_v, din_s.at[dst_v], add=True)
        pltpu.sync_copy(ones_v, dout_s.at[src_v], add=True)
        return 0

    lax.fori_loop(0, CH_ALL, p0, 0)
    plsc.subcore_barrier()

    pltpu.sync_copy(din_s.at[pl.ds(nb, NPT)], b0_v)
    pltpu.sync_copy(dout_s.at[pl.ds(nb, NPT)], b1_v)

    def p1(i, _):
        sl = pl.ds(i * 16, 16)
        di = b0_v[sl]
        do = b1_v[sl]
        rso = _rsqrt16(do)
        b2_v[sl] = di * rso
        b3_v[sl] = _rsqrt16(di)
        b0_v[sl] = rso
        return 0

    lax.fori_loop(0, NPT // 16, p1, 0)
    pltpu.sync_copy(b2_v, g_s.at[pl.ds(nb, NPT)])

    @pl.when(c == 0)
    def _():
        pltpu.sync_copy(b3_v, nin_h.at[pl.ds(nb, NPT)])
        pltpu.sync_copy(b0_v, nout_h.at[pl.ds(nb, NPT)])

    plsc.subcore_barrier()

    def p2(j, _):
        e0 = t * (CH_ALL * CHUNK) + j * CHUNK
        pltpu.sync_copy(src_h.at[pl.ds(e0, CHUNK)], src_v)
        pltpu.sync_copy(dst_h.at[pl.ds(e0, CHUNK)], dst_v)
        pltpu.sync_copy(g_s.at[src_v], vals_v)
        pltpu.sync_copy(vals_v, a_s.at[dst_v], add=True)
        return 0

    lax.fori_loop(0, CH_ALL, p2, 0)
    plsc.subcore_barrier()

    @pl.when(c == 0)
    def _():
        pltpu.sync_copy(a_s.at[pl.ds(nb, NPT)], araw_h.at[pl.ds(nb, NPT)])


def _sc_edge_body(src_h, dst_h, f_h, z2_h, acc2_h,
                  acc_s, src_v, dst_v, rows_v):
    c = lax.axis_index("c")
    t = lax.axis_index("s")
    nb = t * NPT

    pltpu.sync_copy(z2_h.at[pl.ds(nb, NPT), :], acc_s.at[pl.ds(nb, NPT), :])
    plsc.subcore_barrier()

    def p4(j, _):
        e0 = c * EPH + t * (CH_HALF * CHUNK) + j * CHUNK
        pltpu.sync_copy(src_h.at[pl.ds(e0, CHUNK)], src_v)
        pltpu.sync_copy(dst_h.at[pl.ds(e0, CHUNK)], dst_v)
        pltpu.sync_copy(f_h.at[src_v], rows_v)
        pltpu.sync_copy(rows_v, acc_s.at[dst_v], add=True)
        return 0

    lax.fori_loop(0, CH_HALF, p4, 0)
    plsc.subcore_barrier()
    pltpu.sync_copy(acc_s.at[pl.ds(nb, NPT), :],
                    acc2_h.at[c, pl.ds(nb, NPT), :])


def _tc_expand_body(a_ref, ni_ref, no_ref, w_ref, b_ref, f_ref):
    a = a_ref[...] * ni_ref[...]
    f_ref[...] = no_ref[...] * jnp.maximum(a * w_ref[...] + b_ref[...], 0.0)


def _tc_final_body(acc_ref, ni_ref, gid_ref, w2_ref, b2_ref, wc_ref, bc_ref,
                   out_ref):
    agg = (acc_ref[0] + acc_ref[1]) * ni_ref[...]
    h2 = jnp.maximum(
        jnp.dot(agg, w2_ref[...], preferred_element_type=jnp.float32)
        + b2_ref[...], 0.0)
    ids = gid_ref[...]
    garr = lax.broadcasted_iota(jnp.int32, (N_GRAPHS, NP), 0)
    sel = (garr == ids).astype(jnp.float32)
    cnt = jnp.sum(sel, axis=1, keepdims=True)
    hg = jnp.dot(sel, h2, preferred_element_type=jnp.float32)
    hg = hg / jnp.maximum(cnt, 1.0)
    out_ref[...] = (
        jnp.dot(hg, wc_ref[...], preferred_element_type=jnp.float32)
        + bc_ref[...])


_mesh = plsc.VectorSubcoreMesh(core_axis_name="c", subcore_axis_name="s")

_sc_pre = pl.kernel(
    _sc_pre_body,
    out_type=(
        jax.ShapeDtypeStruct((NP,), jnp.float32),
        jax.ShapeDtypeStruct((NP,), jnp.float32),
        jax.ShapeDtypeStruct((NP,), jnp.float32),
    ),
    mesh=_mesh,
    scratch_types=[
        pltpu.VMEM_SHARED((NP,), jnp.float32),
        pltpu.VMEM_SHARED((NP,), jnp.float32),
        pltpu.VMEM_SHARED((NP,), jnp.float32),
        pltpu.VMEM_SHARED((NP,), jnp.float32),
        pltpu.VMEM((CHUNK,), jnp.int32),
        pltpu.VMEM((CHUNK,), jnp.int32),
        pltpu.VMEM((CHUNK,), jnp.float32),
        pltpu.VMEM((CHUNK,), jnp.float32),
        pltpu.VMEM((NPT,), jnp.float32),
        pltpu.VMEM((NPT,), jnp.float32),
        pltpu.VMEM((NPT,), jnp.float32),
        pltpu.VMEM((NPT,), jnp.float32),
    ],
)

_sc_edge = pl.kernel(
    _sc_edge_body,
    out_type=jax.ShapeDtypeStruct((NC, NP, HIDDEN), jnp.float32),
    mesh=_mesh,
    scratch_types=[
        pltpu.VMEM_SHARED((NP, HIDDEN), jnp.float32),
        pltpu.VMEM((CHUNK,), jnp.int32),
        pltpu.VMEM((CHUNK,), jnp.int32),
        pltpu.VMEM((CHUNK, HIDDEN), jnp.float32),
    ],
)

_tc_expand = pl.pallas_call(
    _tc_expand_body,
    out_shape=jax.ShapeDtypeStruct((NP, HIDDEN), jnp.float32),
)

_tc_final = pl.pallas_call(
    _tc_final_body,
    out_shape=jax.ShapeDtypeStruct((N_GRAPHS, N_CLASSES), jnp.float32),
)


@jax.jit
def kernel(edge_index, graph_ids, W1, b1, W2, b2, Wc, bc):
    pad_e = EP - N_EDGES
    src = jnp.concatenate(
        [edge_index[0], jnp.full((pad_e,), NP - 1, jnp.int32)])
    dst = jnp.concatenate(
        [edge_index[1], jnp.full((pad_e,), NP - 1, jnp.int32)])
    gid = jnp.concatenate(
        [graph_ids.astype(jnp.int32),
         jnp.full((NP - N_NODES,), jnp.int32(N_GRAPHS + 7))]).reshape(1, NP)

    z1 = jnp.zeros((NP,), jnp.float32)
    z2 = jnp.zeros((NP, HIDDEN), jnp.float32)

    a_raw, nin, nout = _sc_pre(src, dst, z1)

    f = _tc_expand(a_raw.reshape(NP, 1), nin.reshape(NP, 1),
                   nout.reshape(NP, 1), W1.reshape(1, HIDDEN),
                   b1.reshape(1, HIDDEN))

    acc2 = _sc_edge(src, dst, f, z2)

    out = _tc_final(acc2, nin.reshape(NP, 1), gid, W2,
                    b2.reshape(1, HIDDEN), Wc, bc.reshape(1, N_CLASSES))
    return out


# speedup vs baseline: 15.2902x; 2.8926x over previous
import jax
import jax.numpy as jnp
from jax import lax
from jax.experimental import pallas as pl
from jax.experimental.pallas import tpu as pltpu
from jax.experimental.pallas import tpu_sc as plsc

N_NODES = 10000
N_EDGES = 320000
HIDDEN = 128
N_CLASSES = 10
N_GRAPHS = 128

NP = 10240
NC = 2
NS = 16
NPT = NP // NS
CHUNK = 128
CH_ALL = 160
CH_HALF = 80
NCHUNKS = NC * NS * CH_HALF
EP = NCHUNKS * CHUNK
HHALF = HIDDEN // 2


def _rsqrt16(x):
    xs = jnp.maximum(x, 1.0)
    sc = jnp.where(xs >= 65536.0, 256.0, jnp.where(xs >= 256.0, 16.0, 1.0))
    xp = xs / (sc * sc)
    s = 0.09 * xp + 1.6
    for _ in range(5):
        s = 0.5 * (s + xp / s)
    return jnp.where(x > 0.0, 1.0 / (sc * s), 0.0)


def _pipe_gather_scatter(n, joff, sidx2, didx2, src_tab, dst_tab,
                         buf_a, buf_b, ga, gb, sa, sb):
    def g(j, buf, sem):
        pltpu.async_copy(src_tab.at[sidx2.at[j]], buf, sem)

    def s(j, buf, sem):
        pltpu.async_copy(buf, dst_tab.at[didx2.at[j]], sem, add=True)

    g(joff, buf_a, ga)
    g(joff + 1, buf_b, gb)
    last = joff + n - 1

    def body(p, _):
        j0 = joff + 2 * p
        j1 = j0 + 1
        pltpu.make_async_copy(src_tab.at[sidx2.at[j0]], buf_a, ga).wait()
        s(j0, buf_a, sa)
        pltpu.make_async_copy(src_tab.at[sidx2.at[j1]], buf_b, gb).wait()
        s(j1, buf_b, sb)
        pltpu.make_async_copy(buf_a, dst_tab.at[didx2.at[j0]], sa).wait()
        g(jnp.minimum(j0 + 2, last - 1), buf_a, ga)
        pltpu.make_async_copy(buf_b, dst_tab.at[didx2.at[j1]], sb).wait()
        g(jnp.minimum(j1 + 2, last), buf_b, gb)
        return 0

    lax.fori_loop(0, n // 2, body, 0)
    pltpu.make_async_copy(src_tab.at[sidx2.at[last - 1]], buf_a, ga).wait()
    pltpu.make_async_copy(src_tab.at[sidx2.at[last]], buf_b, gb).wait()


def _sc_pre_body(src2_h, dst2_h, z1_h, araw_h, nin_h, nout_h,
                 din_s, dout_s, g_s, a_s,
                 src2, dst2, ones_v, vals_a, vals_b,
                 b0_v, b1_v, b2_v, b3_v, ga, gb, sa, sb):
    c = lax.axis_index("c")
    t = lax.axis_index("s")
    nb = t * NPT

    pltpu.sync_copy(z1_h.at[pl.ds(nb, NPT)], din_s.at[pl.ds(nb, NPT)])
    pltpu.sync_copy(z1_h.at[pl.ds(nb, NPT)], dout_s.at[pl.ds(nb, NPT)])
    pltpu.sync_copy(z1_h.at[pl.ds(nb, NPT)], a_s.at[pl.ds(nb, NPT)])
    pltpu.sync_copy(src2_h.at[pl.ds(t * CH_ALL, CH_ALL), :], src2)
    pltpu.sync_copy(dst2_h.at[pl.ds(t * CH_ALL, CH_ALL), :], dst2)
    for i in range(CHUNK // 16):
        ones_v[pl.ds(i * 16, 16)] = jnp.ones((16,), jnp.float32)
    plsc.subcore_barrier()

    def p0(j, _):
        @pl.when(j >= 8)
        def _():
            pltpu.make_async_copy(ones_v, din_s.at[dst2.at[0]], sa).wait()
            pltpu.make_async_copy(ones_v, dout_s.at[src2.at[0]], sb).wait()

        pltpu.async_copy(ones_v, din_s.at[dst2.at[j]], sa, add=True)
        pltpu.async_copy(ones_v, dout_s.at[src2.at[j]], sb, add=True)
        return 0

    lax.fori_loop(0, CH_ALL, p0, 0)
    for _ in range(8):
        pltpu.make_async_copy(ones_v, din_s.at[dst2.at[0]], sa).wait()
        pltpu.make_async_copy(ones_v, dout_s.at[src2.at[0]], sb).wait()
    plsc.subcore_barrier()

    pltpu.sync_copy(din_s.at[pl.ds(nb, NPT)], b0_v)
    pltpu.sync_copy(dout_s.at[pl.ds(nb, NPT)], b1_v)

    def p1(i, _):
        sl = pl.ds(i * 16, 16)
        di = b0_v[sl]
        do = b1_v[sl]
        rso = _rsqrt16(do)
        b2_v[sl] = di * rso
        b3_v[sl] = _rsqrt16(di)
        b0_v[sl] = rso
        return 0

    lax.fori_loop(0, NPT // 16, p1, 0)
    pltpu.sync_copy(b2_v, g_s.at[pl.ds(nb, NPT)])

    @pl.when(c == 0)
    def _():
        pltpu.sync_copy(b3_v, nin_h.at[pl.ds(nb, NPT)])
        pltpu.sync_copy(b0_v, nout_h.at[pl.ds(nb, NPT)])

    plsc.subcore_barrier()

    _pipe_gather_scatter(CH_HALF, c * CH_HALF, src2, dst2, g_s, a_s,
                         vals_a, vals_b, ga, gb, sa, sb)
    plsc.subcore_barrier()
    pltpu.sync_copy(a_s.at[pl.ds(nb, NPT)], araw_h.at[c, pl.ds(nb, NPT)])


def _sc_edge_body(src2_h, dst2_h, f2_h, z2_h, acc2_h,
                  acc_s, src2, dst2, rows_a, rows_b, ga, gb, sa, sb):
    c = lax.axis_index("c")
    t = lax.axis_index("s")
    nb = t * NPT

    pltpu.sync_copy(z2_h.at[pl.ds(nb, NPT), :], acc_s.at[pl.ds(nb, NPT), :])
    pltpu.sync_copy(src2_h.at[pl.ds(t * CH_ALL, CH_ALL), :], src2)
    pltpu.sync_copy(dst2_h.at[pl.ds(t * CH_ALL, CH_ALL), :], dst2)
    plsc.subcore_barrier()

    _pipe_gather_scatter(CH_ALL, 0, src2, dst2, f2_h.at[c], acc_s,
                         rows_a, rows_b, ga, gb, sa, sb)
    plsc.subcore_barrier()
    pltpu.sync_copy(acc_s.at[pl.ds(nb, NPT), :],
                    acc2_h.at[c, pl.ds(nb, NPT), :])


def _tc_expand_body(a0_ref, a1_ref, ni_ref, no_ref, w_ref, b_ref, f_ref):
    a = (a0_ref[...] + a1_ref[...]) * ni_ref[...]
    f = no_ref[...] * jnp.maximum(a * w_ref[...] + b_ref[...], 0.0)
    f_ref[0] = f[:, :HHALF]
    f_ref[1] = f[:, HHALF:]


def _tc_final_body(acc_ref, ni_ref, gid_ref, w2_ref, b2_ref, wc_ref, bc_ref,
                   out_ref):
    agg = jnp.concatenate([acc_ref[0], acc_ref[1]], axis=1) * ni_ref[...]
    h2 = jnp.maximum(
        jnp.dot(agg, w2_ref[...], preferred_element_type=jnp.float32)
        + b2_ref[...], 0.0)
    ids = gid_ref[...]
    garr = lax.broadcasted_iota(jnp.int32, (N_GRAPHS, NP), 0)
    sel = (garr == ids).astype(jnp.float32)
    cnt = jnp.sum(sel, axis=1, keepdims=True)
    hg = jnp.dot(sel, h2, preferred_element_type=jnp.float32)
    hg = hg / jnp.maximum(cnt, 1.0)
    out_ref[...] = (
        jnp.dot(hg, wc_ref[...], preferred_element_type=jnp.float32)
        + bc_ref[...])


_mesh = plsc.VectorSubcoreMesh(core_axis_name="c", subcore_axis_name="s")

_sc_pre = pl.kernel(
    _sc_pre_body,
    out_type=(
        jax.ShapeDtypeStruct((NC, NP), jnp.float32),
        jax.ShapeDtypeStruct((NP,), jnp.float32),
        jax.ShapeDtypeStruct((NP,), jnp.float32),
    ),
    mesh=_mesh,
    scratch_types=[
        pltpu.VMEM_SHARED((NP,), jnp.float32),
        pltpu.VMEM_SHARED((NP,), jnp.float32),
        pltpu.VMEM_SHARED((NP,), jnp.float32),
        pltpu.VMEM_SHARED((NP,), jnp.float32),
        pltpu.VMEM((CH_ALL, CHUNK), jnp.int32),
        pltpu.VMEM((CH_ALL, CHUNK), jnp.int32),
        pltpu.VMEM((CHUNK,), jnp.float32),
        pltpu.VMEM((CHUNK,), jnp.float32),
        pltpu.VMEM((CHUNK,), jnp.float32),
        pltpu.VMEM((NPT,), jnp.float32),
        pltpu.VMEM((NPT,), jnp.float32),
        pltpu.VMEM((NPT,), jnp.float32),
        pltpu.VMEM((NPT,), jnp.float32),
        pltpu.SemaphoreType.DMA,
        pltpu.SemaphoreType.DMA,
        pltpu.SemaphoreType.DMA,
        pltpu.SemaphoreType.DMA,
    ],
)

_sc_edge = pl.kernel(
    _sc_edge_body,
    out_type=jax.ShapeDtypeStruct((NC, NP, HHALF), jnp.float32),
    mesh=_mesh,
    compiler_params=pltpu.CompilerParams(use_tc_tiling_on_sc=False),
    scratch_types=[
        pltpu.VMEM_SHARED((NP, HHALF), jnp.float32),
        pltpu.VMEM((CH_ALL, CHUNK), jnp.int32),
        pltpu.VMEM((CH_ALL, CHUNK), jnp.int32),
        pltpu.VMEM((CHUNK, HHALF), jnp.float32),
        pltpu.VMEM((CHUNK, HHALF), jnp.float32),
        pltpu.SemaphoreType.DMA,
        pltpu.SemaphoreType.DMA,
        pltpu.SemaphoreType.DMA,
        pltpu.SemaphoreType.DMA,
    ],
)

_tc_expand = pl.pallas_call(
    _tc_expand_body,
    out_shape=jax.ShapeDtypeStruct((NC, NP, HHALF), jnp.float32),
)

_tc_final = pl.pallas_call(
    _tc_final_body,
    out_shape=jax.ShapeDtypeStruct((N_GRAPHS, N_CLASSES), jnp.float32),
)


@jax.jit
def kernel(edge_index, graph_ids, W1, b1, W2, b2, Wc, bc):
    pad_e = EP - N_EDGES
    spread = (jnp.arange(pad_e, dtype=jnp.int32) % (NP - N_NODES)) + N_NODES
    src2 = jnp.concatenate([edge_index[0], spread]).reshape(NCHUNKS, CHUNK)
    dst2 = jnp.concatenate([edge_index[1], spread]).reshape(NCHUNKS, CHUNK)
    gid = jnp.concatenate(
        [graph_ids.astype(jnp.int32),
         jnp.full((NP - N_NODES,), jnp.int32(N_GRAPHS + 7))]).reshape(1, NP)

    z1 = jnp.zeros((NP,), jnp.float32)
    z2 = jnp.zeros((NP, HHALF), jnp.float32)

    a_raw, nin, nout = _sc_pre(src2, dst2, z1)

    f = _tc_expand(a_raw[0].reshape(NP, 1), a_raw[1].reshape(NP, 1),
                   nin.reshape(NP, 1), nout.reshape(NP, 1),
                   W1.reshape(1, HIDDEN), b1.reshape(1, HIDDEN))

    acc = _sc_edge(src2, dst2, f, z2)

    out = _tc_final(acc, nin.reshape(NP, 1), gid, W2,
                    b2.reshape(1, HIDDEN), Wc, bc.reshape(1, N_CLASSES))
    return out
